# baseline (speedup 1.0000x reference)
import jax
import jax.numpy as jnp
from jax import lax
from jax.experimental import pallas as pl
from jax.experimental.pallas import tpu as pltpu
from jax.experimental.pallas import tpu_sc as plsc

_NC = 2
_NS = 16
_CH = 128
_SL = 2


def _make_scatter(tbl_rows, dhalf, rpw, n_acc):
    stripe = n_acc // _NS
    mesh = plsc.VectorSubcoreMesh(core_axis_name="c", subcore_axis_name="s")

    out_type = jax.ShapeDtypeStruct((_NC, n_acc, dhalf), jnp.float32)
    scratch = (
        pltpu.VMEM((rpw, _SL * _CH), jnp.int32),
        pltpu.VMEM((rpw, _SL * _CH), jnp.int32),
        pltpu.VMEM((_SL * _CH, dhalf), jnp.float32),
        pltpu.VMEM((_SL * _CH, dhalf), jnp.float32),
        pltpu.VMEM_SHARED((n_acc, dhalf), jnp.float32),
        pltpu.SemaphoreType.DMA,
        pltpu.SemaphoreType.DMA,
    )

    def body(table, gidx_h, dst_h, zeros_h, out,
             gidx_v, dst_v, buf0, buf1, acc, sem0, sem1):
        cid = lax.axis_index("c")
        sid = lax.axis_index("s")

        pltpu.sync_copy(zeros_h, acc.at[pl.ds(sid * stripe, stripe)])

        pltpu.sync_copy(gidx_h.at[cid * _NS + sid], gidx_v)
        pltpu.sync_copy(dst_h.at[sid], dst_v)

        def fire(j, buf, sem):
            pltpu.async_copy(table.at[gidx_v.at[j]], buf, sem)

        def wait(j, buf, sem):
            pltpu.make_async_copy(table.at[gidx_v.at[j]], buf, sem).wait()

        def scat(j, buf):
            pltpu.sync_copy(buf, acc.at[dst_v.at[j]], add=True)

        fire(0, buf0, sem0)
        plsc.subcore_barrier()

        @pl.loop(0, (rpw - 1) // 2)
        def _(t):
            j0 = 2 * t
            fire(j0 + 1, buf1, sem1)
            wait(j0, buf0, sem0)
            scat(j0, buf0)
            fire(j0 + 2, buf0, sem0)
            wait(j0 + 1, buf1, sem1)
            scat(j0 + 1, buf1)

        wait(rpw - 1, buf0, sem0)
        scat(rpw - 1, buf0)

        plsc.subcore_barrier()
        pltpu.sync_copy(acc.at[pl.ds(sid * stripe, stripe)],
                        out.at[cid, pl.ds(sid * stripe, stripe)])

    return pl.kernel(body, out_type=out_type, mesh=mesh,
                     scratch_types=scratch,
                     compiler_params=pltpu.CompilerParams(
                         use_tc_tiling_on_sc=False))


def _make_deg(rpw, n_acc):
    stripe = n_acc // _NS
    mesh = plsc.VectorSubcoreMesh(core_axis_name="c", subcore_axis_name="s")

    out_type = jax.ShapeDtypeStruct((_NC, n_acc, 16), jnp.float32)
    scratch = (
        pltpu.VMEM((rpw, _SL * _CH), jnp.int32),
        pltpu.VMEM((_SL * _CH, 16), jnp.float32),
        pltpu.VMEM_SHARED((n_acc, 16), jnp.float32),
    )

    def body(dst_h, zeros16_h, ones_h, out, dst_v, ones_v, degacc):
        cid = lax.axis_index("c")
        sid = lax.axis_index("s")
        pltpu.sync_copy(zeros16_h, degacc.at[pl.ds(sid * stripe, stripe)])
        pltpu.sync_copy(ones_h, ones_v)
        pltpu.sync_copy(dst_h.at[sid], dst_v)
        plsc.subcore_barrier()

        @pl.loop(0, rpw)
        def _(j):
            pltpu.sync_copy(ones_v, degacc.at[dst_v.at[j]], add=True)

        plsc.subcore_barrier()
        pltpu.sync_copy(degacc.at[pl.ds(sid * stripe, stripe)],
                        out.at[cid, pl.ds(sid * stripe, stripe)])

    return pl.kernel(body, out_type=out_type, mesh=mesh,
                     scratch_types=scratch,
                     compiler_params=pltpu.CompilerParams(
                         use_tc_tiling_on_sc=False))


def _mm_body(x_ref, w_ref, o_ref):
    res = jnp.dot(x_ref[...], w_ref[0], preferred_element_type=jnp.float32)
    dhalf = res.shape[1] // 2
    o_ref[0, 0] = res[:, :dhalf]
    o_ref[1, 0] = res[:, dhalf:]


def _matmul_stack(x, wcat, bn):
    k, din, dh = wcat.shape
    n = x.shape[0]
    return pl.pallas_call(
        _mm_body,
        grid=(k, n // bn),
        in_specs=[pl.BlockSpec((bn, din), lambda r, i: (i, 0)),
                  pl.BlockSpec((1, din, dh), lambda r, i: (r, 0, 0))],
        out_specs=pl.BlockSpec((2, 1, bn, dh // 2), lambda r, i: (0, r, i, 0)),
        out_shape=jax.ShapeDtypeStruct((2, k, n, dh // 2), jnp.float32),
    )(x, wcat)


def _relu_body(y0_ref, y1_ref, p0_ref, p1_ref, d0_ref, o_ref):
    inv = 1.0 / jnp.maximum(d0_ref[0][:, :1], 1.0)
    o_ref[0] = jnp.maximum((y0_ref[0, 0] + p0_ref[0]) * inv, 0.0)
    o_ref[1] = jnp.maximum((y1_ref[0, 0] + p1_ref[0]) * inv, 0.0)


def _relu_deg(ycat, p, dg, n, dhalf, bn):
    return pl.pallas_call(
        _relu_body,
        grid=(n // bn,),
        in_specs=[pl.BlockSpec((1, 1, bn, dhalf), lambda i: (0, 0, i, 0)),
                  pl.BlockSpec((1, 1, bn, dhalf), lambda i: (1, 0, i, 0)),
                  pl.BlockSpec((1, bn, dhalf), lambda i: (0, i, 0)),
                  pl.BlockSpec((1, bn, dhalf), lambda i: (1, i, 0)),
                  pl.BlockSpec((1, bn, 16), lambda i: (0, i, 0))],
        out_specs=pl.BlockSpec((2, bn, dhalf), lambda i: (0, i, 0)),
        out_shape=jax.ShapeDtypeStruct((2, n, dhalf), jnp.float32),
    )(ycat, ycat, p, p, dg)


def _group_mm_body(h_ref, p0_ref, p1_ref, d0_ref, pwt_ref, pb_ref,
                   w_ref, o_ref):
    inv = 1.0 / jnp.maximum(d0_ref[0][:, :1], 1.0)
    h = jnp.concatenate([h_ref[0], h_ref[1]], axis=1)
    agg = jnp.concatenate([p0_ref[0], p1_ref[0]], axis=1) * inv
    g = (h + pb_ref[...]
         + jnp.dot(agg, pwt_ref[...], preferred_element_type=jnp.float32))
    res = jnp.dot(g, w_ref[0], preferred_element_type=jnp.float32)
    dhalf = res.shape[1] // 2
    o_ref[0, 0] = res[:, :dhalf]
    o_ref[1, 0] = res[:, dhalf:]


def _group_mm(h, p, dg, pwt, pb, wcat, n, dh, bn):
    k = wcat.shape[0]
    dhalf = dh // 2
    return pl.pallas_call(
        _group_mm_body,
        grid=(k, n // bn),
        in_specs=[pl.BlockSpec((2, bn, dhalf), lambda r, i: (0, i, 0)),
                  pl.BlockSpec((1, bn, dhalf), lambda r, i: (0, i, 0)),
                  pl.BlockSpec((1, bn, dhalf), lambda r, i: (1, i, 0)),
                  pl.BlockSpec((1, bn, 16), lambda r, i: (0, i, 0)),
                  pl.BlockSpec((dh, dh), lambda r, i: (0, 0)),
                  pl.BlockSpec((1, dh), lambda r, i: (0, 0)),
                  pl.BlockSpec((1, dh, dh), lambda r, i: (r, 0, 0))],
        out_specs=pl.BlockSpec((2, 1, bn, dhalf), lambda r, i: (0, r, i, 0)),
        out_shape=jax.ShapeDtypeStruct((2, k, n, dhalf), jnp.float32),
    )(h, p, p, dg, pwt, pb, wcat)


def _final_body(h_ref, p0_ref, p1_ref, d0_ref, pwt_ref, pb_ref,
                owt_ref, ob_ref, o_ref):
    inv = 1.0 / jnp.maximum(d0_ref[0][:, :1], 1.0)
    h = jnp.concatenate([h_ref[0], h_ref[1]], axis=1)
    agg = jnp.concatenate([p0_ref[0], p1_ref[0]], axis=1) * inv
    g = (h + pb_ref[...]
         + jnp.dot(agg, pwt_ref[...], preferred_element_type=jnp.float32))
    o_ref[...] = (jnp.dot(g, owt_ref[...], preferred_element_type=jnp.float32)
                  + ob_ref[...])


def _final(h, p, dg, pwt, pb, owt, ob, n, dh, dout, bn):
    dhalf = dh // 2
    return pl.pallas_call(
        _final_body,
        grid=(n // bn,),
        in_specs=[pl.BlockSpec((2, bn, dhalf), lambda i: (0, i, 0)),
                  pl.BlockSpec((1, bn, dhalf), lambda i: (0, i, 0)),
                  pl.BlockSpec((1, bn, dhalf), lambda i: (1, i, 0)),
                  pl.BlockSpec((1, bn, 16), lambda i: (0, i, 0)),
                  pl.BlockSpec((dh, dh), lambda i: (0, 0)),
                  pl.BlockSpec((1, dh), lambda i: (0, 0)),
                  pl.BlockSpec((dh, dout), lambda i: (0, 0)),
                  pl.BlockSpec((1, dout), lambda i: (0, 0))],
        out_specs=pl.BlockSpec((bn, dout), lambda i: (i, 0)),
        out_shape=jax.ShapeDtypeStruct((n, dout), jnp.float32),
    )(h, p, p, dg, pwt, pb, owt, ob)


def kernel(x, edge_index, edge_type, W1, W0_1, alpha1, proj1_W, proj1_b,
           W2, W0_2, alpha2, proj2_W, proj2_b, out_W, out_b):
    n, din = x.shape
    e = edge_index.shape[1]
    r = W1.shape[0]
    dh = W1.shape[2]
    dhalf = dh // 2
    dout = out_W.shape[0]
    bn = 1000
    k = r + 1

    rpw = -(-e // (_NS * _CH * _SL))
    if rpw % 2 == 0:
        rpw += 1
    pad = rpw * _NS * _CH * _SL - e

    n_acc = -(-(n + 1) // (_NS * 8)) * (_NS * 8)
    stripe = n_acc // _NS

    src = edge_index[0]
    dst = edge_index[1]
    zpad_i = jnp.zeros((pad,), jnp.int32)
    g1 = jnp.concatenate([(edge_type + 1) * n + src, zpad_i])
    s1 = jnp.concatenate([src, zpad_i])
    gpad_rel = jnp.stack([g1, g1 + k * n]).reshape(2 * _NS, rpw, _SL * _CH)
    gpad_grp = jnp.stack([s1, s1 + n]).reshape(2 * _NS, rpw, _SL * _CH)
    dpad = jnp.concatenate([dst, jnp.full((pad,), n, jnp.int32)]
                           ).reshape(_NS, rpw, _SL * _CH)

    zeros_h = jnp.zeros((stripe, dhalf), jnp.float32)
    zeros16_h = jnp.zeros((stripe, 16), jnp.float32)
    ones_h = jnp.ones((_SL * _CH, 16), jnp.float32)

    scat_rel = _make_scatter(k * n, dhalf, rpw, n_acc)
    scat_grp = _make_scatter(n, dhalf, rpw, n_acc)
    deg_kernel = _make_deg(rpw, n_acc)

    dg = deg_kernel(dpad, zeros16_h, ones_h)

    wcat1 = jnp.concatenate([W0_1[None], W1], axis=0)
    ycat1 = _matmul_stack(x, wcat1, bn)
    p_rel1 = scat_rel(ycat1.reshape(2 * k * n, dhalf), gpad_rel, dpad,
                      zeros_h)
    h1 = _relu_deg(ycat1, p_rel1, dg, n, dhalf, bn)

    p_grp1 = scat_grp(h1.reshape(2 * n, dhalf), gpad_grp, dpad, zeros_h)
    wcat2 = jnp.concatenate([W0_2[None], W2], axis=0)
    pwt1 = (alpha1 * proj1_W.T).astype(jnp.float32)
    pb1 = (alpha1 * proj1_b)[None].astype(jnp.float32)
    ycat2 = _group_mm(h1, p_grp1, dg, pwt1, pb1, wcat2, n, dh, bn)

    p_rel2 = scat_rel(ycat2.reshape(2 * k * n, dhalf), gpad_rel, dpad,
                      zeros_h)
    h2 = _relu_deg(ycat2, p_rel2, dg, n, dhalf, bn)

    p_grp2 = scat_grp(h2.reshape(2 * n, dhalf), gpad_grp, dpad, zeros_h)
    pwt2 = (alpha2 * proj2_W.T).astype(jnp.float32)
    pb2 = (alpha2 * proj2_b)[None].astype(jnp.float32)
    logits = _final(h2, p_grp2, dg, pwt2, pb2, out_W.T.astype(jnp.float32),
                    out_b[None].astype(jnp.float32), n, dh, dout, bn)
    return logits


# speedup vs baseline: 1.1382x; 1.1382x over previous
import jax
import jax.numpy as jnp
from jax import lax
from jax.experimental import pallas as pl
from jax.experimental.pallas import tpu as pltpu
from jax.experimental.pallas import tpu_sc as plsc

_NC = 2
_NS = 16
_CH = 128
_SL = 1


def _make_scatter(tbl_rows, dhalf, rpw, n_acc):
    stripe = n_acc // _NS
    mesh = plsc.VectorSubcoreMesh(core_axis_name="c", subcore_axis_name="s")

    out_type = jax.ShapeDtypeStruct((_NC, n_acc, dhalf), jnp.float32)
    scratch = (
        pltpu.VMEM((rpw, _SL * _CH), jnp.int32),
        pltpu.VMEM((rpw, _SL * _CH), jnp.int32),
        pltpu.VMEM((_SL * _CH, dhalf), jnp.float32),
        pltpu.VMEM((_SL * _CH, dhalf), jnp.float32),
        pltpu.VMEM_SHARED((n_acc, dhalf), jnp.float32),
        pltpu.SemaphoreType.DMA,
        pltpu.SemaphoreType.DMA,
    )

    def body(table, gidx_h, dst_h, zeros_h, out,
             gidx_v, dst_v, buf0, buf1, acc, sem0, sem1):
        cid = lax.axis_index("c")
        sid = lax.axis_index("s")

        pltpu.sync_copy(zeros_h, acc.at[pl.ds(sid * stripe, stripe)])

        pltpu.sync_copy(gidx_h.at[cid * _NS + sid], gidx_v)
        pltpu.sync_copy(dst_h.at[sid], dst_v)

        def fire(j, buf, sem):
            pltpu.async_copy(table.at[gidx_v.at[j]], buf, sem)

        def wait(j, buf, sem):
            pltpu.make_async_copy(table.at[gidx_v.at[j]], buf, sem).wait()

        def scat(j, buf):
            pltpu.sync_copy(buf, acc.at[dst_v.at[j]], add=True)

        fire(0, buf0, sem0)
        plsc.subcore_barrier()

        @pl.loop(0, (rpw - 1) // 2)
        def _(t):
            j0 = 2 * t
            fire(j0 + 1, buf1, sem1)
            wait(j0, buf0, sem0)
            scat(j0, buf0)
            fire(j0 + 2, buf0, sem0)
            wait(j0 + 1, buf1, sem1)
            scat(j0 + 1, buf1)

        wait(rpw - 1, buf0, sem0)
        scat(rpw - 1, buf0)

        plsc.subcore_barrier()
        pltpu.sync_copy(acc.at[pl.ds(sid * stripe, stripe)],
                        out.at[cid, pl.ds(sid * stripe, stripe)])

    return pl.kernel(body, out_type=out_type, mesh=mesh,
                     scratch_types=scratch,
                     compiler_params=pltpu.CompilerParams(
                         use_tc_tiling_on_sc=False))


def _make_deg(rpw, n_acc):
    stripe = n_acc // _NS
    mesh = plsc.VectorSubcoreMesh(core_axis_name="c", subcore_axis_name="s")

    out_type = jax.ShapeDtypeStruct((_NC, n_acc, 16), jnp.float32)
    scratch = (
        pltpu.VMEM((rpw, _SL * _CH), jnp.int32),
        pltpu.VMEM((_SL * _CH, 16), jnp.float32),
        pltpu.VMEM_SHARED((n_acc, 16), jnp.float32),
    )

    def body(dst_h, zeros16_h, ones_h, out, dst_v, ones_v, degacc):
        cid = lax.axis_index("c")
        sid = lax.axis_index("s")
        pltpu.sync_copy(zeros16_h, degacc.at[pl.ds(sid * stripe, stripe)])
        pltpu.sync_copy(ones_h, ones_v)
        pltpu.sync_copy(dst_h.at[sid], dst_v)
        plsc.subcore_barrier()

        @pl.loop(0, rpw)
        def _(j):
            pltpu.sync_copy(ones_v, degacc.at[dst_v.at[j]], add=True)

        plsc.subcore_barrier()
        pltpu.sync_copy(degacc.at[pl.ds(sid * stripe, stripe)],
                        out.at[cid, pl.ds(sid * stripe, stripe)])

    return pl.kernel(body, out_type=out_type, mesh=mesh,
                     scratch_types=scratch,
                     compiler_params=pltpu.CompilerParams(
                         use_tc_tiling_on_sc=False))


def _mm_body(x_ref, w_ref, o_ref):
    res = jnp.dot(x_ref[...], w_ref[0], preferred_element_type=jnp.float32)
    dhalf = res.shape[1] // 2
    o_ref[0, 0] = res[:, :dhalf]
    o_ref[1, 0] = res[:, dhalf:]


def _matmul_stack(x, wcat, bn):
    k, din, dh = wcat.shape
    n = x.shape[0]
    return pl.pallas_call(
        _mm_body,
        grid=(k, n // bn),
        in_specs=[pl.BlockSpec((bn, din), lambda r, i: (i, 0)),
                  pl.BlockSpec((1, din, dh), lambda r, i: (r, 0, 0))],
        out_specs=pl.BlockSpec((2, 1, bn, dh // 2), lambda r, i: (0, r, i, 0)),
        out_shape=jax.ShapeDtypeStruct((2, k, n, dh // 2), jnp.float32),
    )(x, wcat)


def _relu_body(y0_ref, y1_ref, p0_ref, p1_ref, d0_ref, o_ref):
    inv = 1.0 / jnp.maximum(d0_ref[0][:, :1], 1.0)
    o_ref[0] = jnp.maximum((y0_ref[0, 0] + p0_ref[0]) * inv, 0.0)
    o_ref[1] = jnp.maximum((y1_ref[0, 0] + p1_ref[0]) * inv, 0.0)


def _relu_deg(ycat, p, dg, n, dhalf, bn):
    return pl.pallas_call(
        _relu_body,
        grid=(n // bn,),
        in_specs=[pl.BlockSpec((1, 1, bn, dhalf), lambda i: (0, 0, i, 0)),
                  pl.BlockSpec((1, 1, bn, dhalf), lambda i: (1, 0, i, 0)),
                  pl.BlockSpec((1, bn, dhalf), lambda i: (0, i, 0)),
                  pl.BlockSpec((1, bn, dhalf), lambda i: (1, i, 0)),
                  pl.BlockSpec((1, bn, 16), lambda i: (0, i, 0))],
        out_specs=pl.BlockSpec((2, bn, dhalf), lambda i: (0, i, 0)),
        out_shape=jax.ShapeDtypeStruct((2, n, dhalf), jnp.float32),
    )(ycat, ycat, p, p, dg)


def _group_mm_body(h_ref, p0_ref, p1_ref, d0_ref, pwt_ref, pb_ref,
                   w_ref, o_ref):
    inv = 1.0 / jnp.maximum(d0_ref[0][:, :1], 1.0)
    h = jnp.concatenate([h_ref[0], h_ref[1]], axis=1)
    agg = jnp.concatenate([p0_ref[0], p1_ref[0]], axis=1) * inv
    g = (h + pb_ref[...]
         + jnp.dot(agg, pwt_ref[...], preferred_element_type=jnp.float32))
    res = jnp.dot(g, w_ref[0], preferred_element_type=jnp.float32)
    dhalf = res.shape[1] // 2
    o_ref[0, 0] = res[:, :dhalf]
    o_ref[1, 0] = res[:, dhalf:]


def _group_mm(h, p, dg, pwt, pb, wcat, n, dh, bn):
    k = wcat.shape[0]
    dhalf = dh // 2
    return pl.pallas_call(
        _group_mm_body,
        grid=(k, n // bn),
        in_specs=[pl.BlockSpec((2, bn, dhalf), lambda r, i: (0, i, 0)),
                  pl.BlockSpec((1, bn, dhalf), lambda r, i: (0, i, 0)),
                  pl.BlockSpec((1, bn, dhalf), lambda r, i: (1, i, 0)),
                  pl.BlockSpec((1, bn, 16), lambda r, i: (0, i, 0)),
                  pl.BlockSpec((dh, dh), lambda r, i: (0, 0)),
                  pl.BlockSpec((1, dh), lambda r, i: (0, 0)),
                  pl.BlockSpec((1, dh, dh), lambda r, i: (r, 0, 0))],
        out_specs=pl.BlockSpec((2, 1, bn, dhalf), lambda r, i: (0, r, i, 0)),
        out_shape=jax.ShapeDtypeStruct((2, k, n, dhalf), jnp.float32),
    )(h, p, p, dg, pwt, pb, wcat)


def _final_body(h_ref, p0_ref, p1_ref, d0_ref, pwt_ref, pb_ref,
                owt_ref, ob_ref, o_ref):
    inv = 1.0 / jnp.maximum(d0_ref[0][:, :1], 1.0)
    h = jnp.concatenate([h_ref[0], h_ref[1]], axis=1)
    agg = jnp.concatenate([p0_ref[0], p1_ref[0]], axis=1) * inv
    g = (h + pb_ref[...]
         + jnp.dot(agg, pwt_ref[...], preferred_element_type=jnp.float32))
    o_ref[...] = (jnp.dot(g, owt_ref[...], preferred_element_type=jnp.float32)
                  + ob_ref[...])


def _final(h, p, dg, pwt, pb, owt, ob, n, dh, dout, bn):
    dhalf = dh // 2
    return pl.pallas_call(
        _final_body,
        grid=(n // bn,),
        in_specs=[pl.BlockSpec((2, bn, dhalf), lambda i: (0, i, 0)),
                  pl.BlockSpec((1, bn, dhalf), lambda i: (0, i, 0)),
                  pl.BlockSpec((1, bn, dhalf), lambda i: (1, i, 0)),
                  pl.BlockSpec((1, bn, 16), lambda i: (0, i, 0)),
                  pl.BlockSpec((dh, dh), lambda i: (0, 0)),
                  pl.BlockSpec((1, dh), lambda i: (0, 0)),
                  pl.BlockSpec((dh, dout), lambda i: (0, 0)),
                  pl.BlockSpec((1, dout), lambda i: (0, 0))],
        out_specs=pl.BlockSpec((bn, dout), lambda i: (i, 0)),
        out_shape=jax.ShapeDtypeStruct((n, dout), jnp.float32),
    )(h, p, p, dg, pwt, pb, owt, ob)


def kernel(x, edge_index, edge_type, W1, W0_1, alpha1, proj1_W, proj1_b,
           W2, W0_2, alpha2, proj2_W, proj2_b, out_W, out_b):
    n, din = x.shape
    e = edge_index.shape[1]
    r = W1.shape[0]
    dh = W1.shape[2]
    dhalf = dh // 2
    dout = out_W.shape[0]
    bn = 1000
    k = r + 1

    rpw = -(-e // (_NS * _CH * _SL))
    if rpw % 2 == 0:
        rpw += 1
    pad = rpw * _NS * _CH * _SL - e

    n_acc = -(-(n + 1) // (_NS * 8)) * (_NS * 8)
    stripe = n_acc // _NS

    src = edge_index[0]
    dst = edge_index[1]
    zpad_i = jnp.zeros((pad,), jnp.int32)
    g1 = jnp.concatenate([(edge_type + 1) * n + src, zpad_i])
    s1 = jnp.concatenate([src, zpad_i])
    gpad_rel = jnp.stack([g1, g1 + k * n]).reshape(2 * _NS, rpw, _SL * _CH)
    gpad_grp = jnp.stack([s1, s1 + n]).reshape(2 * _NS, rpw, _SL * _CH)
    dpad = jnp.concatenate([dst, jnp.full((pad,), n, jnp.int32)]
                           ).reshape(_NS, rpw, _SL * _CH)

    zeros_h = jnp.zeros((stripe, dhalf), jnp.float32)
    zeros16_h = jnp.zeros((stripe, 16), jnp.float32)
    ones_h = jnp.ones((_SL * _CH, 16), jnp.float32)

    scat_rel = _make_scatter(k * n, dhalf, rpw, n_acc)
    scat_grp = _make_scatter(n, dhalf, rpw, n_acc)
    deg_kernel = _make_deg(rpw, n_acc)

    dg = deg_kernel(dpad, zeros16_h, ones_h)

    wcat1 = jnp.concatenate([W0_1[None], W1], axis=0)
    ycat1 = _matmul_stack(x, wcat1, bn)
    p_rel1 = scat_rel(ycat1.reshape(2 * k * n, dhalf), gpad_rel, dpad,
                      zeros_h)
    h1 = _relu_deg(ycat1, p_rel1, dg, n, dhalf, bn)

    p_grp1 = scat_grp(h1.reshape(2 * n, dhalf), gpad_grp, dpad, zeros_h)
    wcat2 = jnp.concatenate([W0_2[None], W2], axis=0)
    pwt1 = (alpha1 * proj1_W.T).astype(jnp.float32)
    pb1 = (alpha1 * proj1_b)[None].astype(jnp.float32)
    ycat2 = _group_mm(h1, p_grp1, dg, pwt1, pb1, wcat2, n, dh, bn)

    p_rel2 = scat_rel(ycat2.reshape(2 * k * n, dhalf), gpad_rel, dpad,
                      zeros_h)
    h2 = _relu_deg(ycat2, p_rel2, dg, n, dhalf, bn)

    p_grp2 = scat_grp(h2.reshape(2 * n, dhalf), gpad_grp, dpad, zeros_h)
    pwt2 = (alpha2 * proj2_W.T).astype(jnp.float32)
    pb2 = (alpha2 * proj2_b)[None].astype(jnp.float32)
    logits = _final(h2, p_grp2, dg, pwt2, pb2, out_W.T.astype(jnp.float32),
                    out_b[None].astype(jnp.float32), n, dh, dout, bn)
    return logits


# speedup vs baseline: 1.1861x; 1.0420x over previous
import jax
import jax.numpy as jnp
from jax import lax
from jax.experimental import pallas as pl
from jax.experimental.pallas import tpu as pltpu
from jax.experimental.pallas import tpu_sc as plsc

_NC = 2
_NS = 16
_CH = 128
_SL = 1


def _make_scatter(tbl_rows, dhalf, rpw, n_acc):
    stripe = n_acc // _NS
    mesh = plsc.VectorSubcoreMesh(core_axis_name="c", subcore_axis_name="s")

    out_type = jax.ShapeDtypeStruct((_NC, n_acc, dhalf), jnp.float32)
    scratch = (
        pltpu.VMEM((rpw, _SL * _CH), jnp.int32),
        pltpu.VMEM((rpw, _SL * _CH), jnp.int32),
        pltpu.VMEM((_SL * _CH, dhalf), jnp.float32),
        pltpu.VMEM((_SL * _CH, dhalf), jnp.float32),
        pltpu.VMEM_SHARED((n_acc, dhalf), jnp.float32),
        pltpu.SemaphoreType.DMA,
        pltpu.SemaphoreType.DMA,
    )

    def body(table, gidx_h, dst_h, zeros_h, out,
             gidx_v, dst_v, buf0, buf1, acc, sem0, sem1):
        cid = lax.axis_index("c")
        sid = lax.axis_index("s")

        pltpu.sync_copy(zeros_h, acc.at[pl.ds(sid * stripe, stripe)])

        pltpu.sync_copy(gidx_h.at[cid * _NS + sid], gidx_v)
        pltpu.sync_copy(dst_h.at[sid], dst_v)

        def fire(j, buf, sem):
            pltpu.async_copy(table.at[gidx_v.at[j]], buf, sem)

        def wait(j, buf, sem):
            pltpu.make_async_copy(table.at[gidx_v.at[j]], buf, sem).wait()

        def scat(j, buf):
            pltpu.sync_copy(buf, acc.at[dst_v.at[j]], add=True)

        fire(0, buf0, sem0)
        plsc.subcore_barrier()

        @pl.loop(0, (rpw - 1) // 2)
        def _(t):
            j0 = 2 * t
            fire(j0 + 1, buf1, sem1)
            wait(j0, buf0, sem0)
            scat(j0, buf0)
            fire(j0 + 2, buf0, sem0)
            wait(j0 + 1, buf1, sem1)
            scat(j0 + 1, buf1)

        wait(rpw - 1, buf0, sem0)
        scat(rpw - 1, buf0)

        plsc.subcore_barrier()
        pltpu.sync_copy(acc.at[pl.ds(sid * stripe, stripe)],
                        out.at[cid, pl.ds(sid * stripe, stripe)])

    return pl.kernel(body, out_type=out_type, mesh=mesh,
                     scratch_types=scratch,
                     compiler_params=pltpu.CompilerParams(
                         use_tc_tiling_on_sc=False))


def _make_deg(rpw, n_acc):
    stripe = n_acc // _NS
    mesh = plsc.VectorSubcoreMesh(core_axis_name="c", subcore_axis_name="s")

    out_type = jax.ShapeDtypeStruct((_NC, n_acc, 16), jnp.float32)
    scratch = (
        pltpu.VMEM((rpw, _SL * _CH), jnp.int32),
        pltpu.VMEM((_SL * _CH, 16), jnp.float32),
        pltpu.VMEM_SHARED((n_acc, 16), jnp.float32),
    )

    def body(dst_h, zeros16_h, ones_h, out, dst_v, ones_v, degacc):
        cid = lax.axis_index("c")
        sid = lax.axis_index("s")
        pltpu.sync_copy(zeros16_h, degacc.at[pl.ds(sid * stripe, stripe)])
        pltpu.sync_copy(ones_h, ones_v)
        pltpu.sync_copy(dst_h.at[sid], dst_v)
        plsc.subcore_barrier()

        @pl.loop(0, rpw)
        def _(j):
            pltpu.sync_copy(ones_v, degacc.at[dst_v.at[j]], add=True)

        plsc.subcore_barrier()
        pltpu.sync_copy(degacc.at[pl.ds(sid * stripe, stripe)],
                        out.at[cid, pl.ds(sid * stripe, stripe)])

    return pl.kernel(body, out_type=out_type, mesh=mesh,
                     scratch_types=scratch,
                     compiler_params=pltpu.CompilerParams(
                         use_tc_tiling_on_sc=False))


def _mm_body(x_ref, w_ref, o_ref):
    res = jnp.dot(x_ref[...], w_ref[0], preferred_element_type=jnp.float32)
    dhalf = res.shape[1] // 2
    o_ref[0, 0] = res[:, :dhalf]
    o_ref[1, 0] = res[:, dhalf:]


def _matmul_stack(x, wcat, bn):
    k, din, dh = wcat.shape
    n = x.shape[0]
    return pl.pallas_call(
        _mm_body,
        grid=(k, n // bn),
        in_specs=[pl.BlockSpec((bn, din), lambda r, i: (i, 0)),
                  pl.BlockSpec((1, din, dh), lambda r, i: (r, 0, 0))],
        out_specs=pl.BlockSpec((2, 1, bn, dh // 2), lambda r, i: (0, r, i, 0)),
        out_shape=jax.ShapeDtypeStruct((2, k, n, dh // 2), jnp.float32),
    )(x, wcat)


def _relu_body(y0_ref, y1_ref, p0_ref, p1_ref, d0_ref, o_ref):
    inv = 1.0 / jnp.maximum(d0_ref[0][:, :1], 1.0)
    o_ref[0] = jnp.maximum((y0_ref[0, 0] + p0_ref[0]) * inv, 0.0)
    o_ref[1] = jnp.maximum((y1_ref[0, 0] + p1_ref[0]) * inv, 0.0)


def _relu_deg(ycat, p, dg, n, dhalf, bn):
    return pl.pallas_call(
        _relu_body,
        grid=(n // bn,),
        in_specs=[pl.BlockSpec((1, 1, bn, dhalf), lambda i: (0, 0, i, 0)),
                  pl.BlockSpec((1, 1, bn, dhalf), lambda i: (1, 0, i, 0)),
                  pl.BlockSpec((1, bn, dhalf), lambda i: (0, i, 0)),
                  pl.BlockSpec((1, bn, dhalf), lambda i: (1, i, 0)),
                  pl.BlockSpec((1, bn, 16), lambda i: (0, i, 0))],
        out_specs=pl.BlockSpec((2, bn, dhalf), lambda i: (0, i, 0)),
        out_shape=jax.ShapeDtypeStruct((2, n, dhalf), jnp.float32),
    )(ycat, ycat, p, p, dg)


def _group_mm_body(h_ref, p0_ref, p1_ref, d0_ref, pwt_ref, pb_ref,
                   w_ref, o_ref):
    inv = 1.0 / jnp.maximum(d0_ref[0][:, :1], 1.0)
    h = jnp.concatenate([h_ref[0], h_ref[1]], axis=1)
    agg = jnp.concatenate([p0_ref[0], p1_ref[0]], axis=1) * inv
    g = (h + pb_ref[...]
         + jnp.dot(agg, pwt_ref[...], preferred_element_type=jnp.float32))
    res = jnp.dot(g, w_ref[0], preferred_element_type=jnp.float32)
    dhalf = res.shape[1] // 2
    o_ref[0, 0] = res[:, :dhalf]
    o_ref[1, 0] = res[:, dhalf:]


def _group_mm(h, p, dg, pwt, pb, wcat, n, dh, bn):
    k = wcat.shape[0]
    dhalf = dh // 2
    return pl.pallas_call(
        _group_mm_body,
        grid=(k, n // bn),
        in_specs=[pl.BlockSpec((2, bn, dhalf), lambda r, i: (0, i, 0)),
                  pl.BlockSpec((1, bn, dhalf), lambda r, i: (0, i, 0)),
                  pl.BlockSpec((1, bn, dhalf), lambda r, i: (1, i, 0)),
                  pl.BlockSpec((1, bn, 16), lambda r, i: (0, i, 0)),
                  pl.BlockSpec((dh, dh), lambda r, i: (0, 0)),
                  pl.BlockSpec((1, dh), lambda r, i: (0, 0)),
                  pl.BlockSpec((1, dh, dh), lambda r, i: (r, 0, 0))],
        out_specs=pl.BlockSpec((2, 1, bn, dhalf), lambda r, i: (0, r, i, 0)),
        out_shape=jax.ShapeDtypeStruct((2, k, n, dhalf), jnp.float32),
    )(h, p, p, dg, pwt, pb, wcat)


def _final_body(h_ref, p0_ref, p1_ref, d0_ref, pwt_ref, pb_ref,
                owt_ref, ob_ref, o_ref):
    inv = 1.0 / jnp.maximum(d0_ref[0][:, :1], 1.0)
    h = jnp.concatenate([h_ref[0], h_ref[1]], axis=1)
    agg = jnp.concatenate([p0_ref[0], p1_ref[0]], axis=1) * inv
    g = (h + pb_ref[...]
         + jnp.dot(agg, pwt_ref[...], preferred_element_type=jnp.float32))
    o_ref[...] = (jnp.dot(g, owt_ref[...], preferred_element_type=jnp.float32)
                  + ob_ref[...])


def _final(h, p, dg, pwt, pb, owt, ob, n, dh, dout, bn):
    dhalf = dh // 2
    return pl.pallas_call(
        _final_body,
        grid=(n // bn,),
        in_specs=[pl.BlockSpec((2, bn, dhalf), lambda i: (0, i, 0)),
                  pl.BlockSpec((1, bn, dhalf), lambda i: (0, i, 0)),
                  pl.BlockSpec((1, bn, dhalf), lambda i: (1, i, 0)),
                  pl.BlockSpec((1, bn, 16), lambda i: (0, i, 0)),
                  pl.BlockSpec((dh, dh), lambda i: (0, 0)),
                  pl.BlockSpec((1, dh), lambda i: (0, 0)),
                  pl.BlockSpec((dh, dout), lambda i: (0, 0)),
                  pl.BlockSpec((1, dout), lambda i: (0, 0))],
        out_specs=pl.BlockSpec((bn, dout), lambda i: (i, 0)),
        out_shape=jax.ShapeDtypeStruct((n, dout), jnp.float32),
    )(h, p, p, dg, pwt, pb, owt, ob)


def kernel(x, edge_index, edge_type, W1, W0_1, alpha1, proj1_W, proj1_b,
           W2, W0_2, alpha2, proj2_W, proj2_b, out_W, out_b):
    n, din = x.shape
    e = edge_index.shape[1]
    r = W1.shape[0]
    dh = W1.shape[2]
    dhalf = dh // 2
    dout = out_W.shape[0]
    bn = 2000
    k = r + 1

    rpw = -(-e // (_NS * _CH * _SL))
    if rpw % 2 == 0:
        rpw += 1
    pad = rpw * _NS * _CH * _SL - e

    n_acc = -(-(n + 1) // (_NS * 8)) * (_NS * 8)
    stripe = n_acc // _NS

    src = edge_index[0]
    dst = edge_index[1]
    zpad_i = jnp.zeros((pad,), jnp.int32)
    g1 = jnp.concatenate([(edge_type + 1) * n + src, zpad_i])
    s1 = jnp.concatenate([src, zpad_i])
    gpad_rel = jnp.stack([g1, g1 + k * n]).reshape(2 * _NS, rpw, _SL * _CH)
    gpad_grp = jnp.stack([s1, s1 + n]).reshape(2 * _NS, rpw, _SL * _CH)
    dpad = jnp.concatenate([dst, jnp.full((pad,), n, jnp.int32)]
                           ).reshape(_NS, rpw, _SL * _CH)

    zeros_h = jnp.zeros((stripe, dhalf), jnp.float32)
    zeros16_h = jnp.zeros((stripe, 16), jnp.float32)
    ones_h = jnp.ones((_SL * _CH, 16), jnp.float32)

    scat_rel = _make_scatter(k * n, dhalf, rpw, n_acc)
    scat_grp = _make_scatter(n, dhalf, rpw, n_acc)
    deg_kernel = _make_deg(rpw, n_acc)

    dg = deg_kernel(dpad, zeros16_h, ones_h)

    wcat1 = jnp.concatenate([W0_1[None], W1], axis=0)
    ycat1 = _matmul_stack(x, wcat1, bn)
    p_rel1 = scat_rel(ycat1.reshape(2 * k * n, dhalf), gpad_rel, dpad,
                      zeros_h)
    h1 = _relu_deg(ycat1, p_rel1, dg, n, dhalf, bn)

    p_grp1 = scat_grp(h1.reshape(2 * n, dhalf), gpad_grp, dpad, zeros_h)
    wcat2 = jnp.concatenate([W0_2[None], W2], axis=0)
    pwt1 = (alpha1 * proj1_W.T).astype(jnp.float32)
    pb1 = (alpha1 * proj1_b)[None].astype(jnp.float32)
    ycat2 = _group_mm(h1, p_grp1, dg, pwt1, pb1, wcat2, n, dh, bn)

    p_rel2 = scat_rel(ycat2.reshape(2 * k * n, dhalf), gpad_rel, dpad,
                      zeros_h)
    h2 = _relu_deg(ycat2, p_rel2, dg, n, dhalf, bn)

    p_grp2 = scat_grp(h2.reshape(2 * n, dhalf), gpad_grp, dpad, zeros_h)
    pwt2 = (alpha2 * proj2_W.T).astype(jnp.float32)
    pb2 = (alpha2 * proj2_b)[None].astype(jnp.float32)
    logits = _final(h2, p_grp2, dg, pwt2, pb2, out_W.T.astype(jnp.float32),
                    out_b[None].astype(jnp.float32), n, dh, dout, bn)
    return logits
